# baseline (speedup 1.0000x reference)
import jax
import jax.numpy as jnp
from jax import lax
from jax.experimental import pallas as pl
from jax.experimental.pallas import tpu as pltpu
from jax.experimental.pallas import tpu_sc as plsc

_UN = 100000
_IN = 100000
_T = 26
_D = 16
_B = 4096
_ROW = _T * _D
_FB = _UN * _T // _D

_NC = 2
_NS = 16
_NW = _NC * _NS
_EPW = _B // _NW
_CH = 32
_NCH = _EPW // _CH
_NG = _CH // _D


def _body(user_ref, item_ref, uemb_ref, iemb_ref,
          upb_ref, ipb_ref, gpb_ref, unb_ref, inb_ref, gnb_ref,
          uc_ref, ic_ref, gc_ref, out_ref, *scr):
  nper = 14
  sets = [scr[:nper], scr[nper:2 * nper]]
  gpb_v, gnb_v, gc_v, dots, outv, sem0, sem1 = scr[2 * nper:]
  sems = [sem0, sem1]

  cid = lax.axis_index("c")
  sid = lax.axis_index("s")
  wid = sid * _NC + cid
  base = wid * _EPW

  pltpu.sync_copy(gpb_ref, gpb_v.at[pl.ds(0, _T)])
  pltpu.sync_copy(gnb_ref, gnb_v.at[pl.ds(0, _T)])
  pltpu.sync_copy(gc_ref, gc_v.at[pl.ds(0, _T)])

  lane = lax.iota(jnp.int32, _D)
  lane3 = lane * 3
  lane32 = lane * 32
  lane48 = lane * 48
  lane15 = lane == 15

  descs = [None, None]

  def fire(c):
    s = c % 2
    (uidx, iidx, offu, offi, b3u, b3i, urow, irow,
     upb, ipb, unb, inb, uc, ic) = sets[s]
    pltpu.sync_copy(user_ref.at[pl.ds(base + c * _CH, _CH)], uidx)
    pltpu.sync_copy(item_ref.at[pl.ds(base + c * _CH, _CH)], iidx)
    for k in range(_NG):
      pos = lane3 + (k * _D * 3)
      for vec, offbuf, b3buf in ((uidx, offu, b3u), (iidx, offi, b3i)):
        w = vec[pl.ds(k * _D, _D)] * _T
        r0 = lax.shift_right_logical(w, 4)
        offbuf[pl.ds(k * _D, _D)] = jnp.bitwise_and(w, 15)
        for j in range(3):
          rj = jnp.minimum(r0 + j, _FB - 1)
          plsc.store_scatter(b3buf, [pos + j], rj)
    descs[s] = [
        pltpu.async_copy(uemb_ref.at[uidx], urow, sems[s]),
        pltpu.async_copy(iemb_ref.at[iidx], irow, sems[s]),
        pltpu.async_copy(upb_ref.at[b3u], upb, sems[s]),
        pltpu.async_copy(unb_ref.at[b3u], unb, sems[s]),
        pltpu.async_copy(uc_ref.at[b3u], uc, sems[s]),
        pltpu.async_copy(ipb_ref.at[b3i], ipb, sems[s]),
        pltpu.async_copy(inb_ref.at[b3i], inb, sems[s]),
        pltpu.async_copy(ic_ref.at[b3i], ic, sems[s]),
    ]

  def compute(c):
    s = c % 2
    (uidx, iidx, offu, offi, b3u, b3i, urow, irow,
     upb, ipb, unb, inb, uc, ic) = sets[s]

    def elem(e, carry):
      for t in range(_T):
        u = urow[e, pl.ds(t * _D, _D)]
        w = irow[e, pl.ds(t * _D, _D)]
        prod = plsc.cumsum(u * w)
        slot = e * 32 + jnp.bitwise_and(e + t, 31)
        plsc.store_scatter(dots, [jnp.full((_D,), slot, jnp.int32)], prod,
                           mask=lane15)
      return carry

    lax.fori_loop(0, _CH, elem, 0)

    for g in range(_NG):
      g16 = g * _D
      dbase = lane32 + g16 * 32
      skew = lane + g16
      peru = lane48 + g16 * 48 + offu[pl.ds(g16, _D)]
      peri = lane48 + g16 * 48 + offi[pl.ds(g16, _D)]

      def tag(t, acc):
        tf = jnp.full((_D,), t, jnp.int32)
        d = plsc.load_gather(dots, [dbase + jnp.bitwise_and(skew + t, 31)])
        fu = peru + t
        fi = peri + t
        upb_t = plsc.load_gather(upb, [lax.shift_right_logical(fu, 4),
                                       jnp.bitwise_and(fu, 15)])
        unb_t = plsc.load_gather(unb, [lax.shift_right_logical(fu, 4),
                                       jnp.bitwise_and(fu, 15)])
        uc_t = plsc.load_gather(uc, [lax.shift_right_logical(fu, 4),
                                     jnp.bitwise_and(fu, 15)])
        ipb_t = plsc.load_gather(ipb, [lax.shift_right_logical(fi, 4),
                                       jnp.bitwise_and(fi, 15)])
        inb_t = plsc.load_gather(inb, [lax.shift_right_logical(fi, 4),
                                       jnp.bitwise_and(fi, 15)])
        ic_t = plsc.load_gather(ic, [lax.shift_right_logical(fi, 4),
                                     jnp.bitwise_and(fi, 15)])
        gp_t = plsc.load_gather(gpb_v, [tf])
        gn_t = plsc.load_gather(gnb_v, [tf])
        gc_t = plsc.load_gather(gc_v, [tf])

        one = jnp.float32(1.0)
        pos = one / (one + jnp.exp(-(d + upb_t + ipb_t + gp_t)))
        neg = one / (one + jnp.exp(-(d + unb_t + inb_t + gn_t)))
        score = (pos + neg) * jnp.float32(0.5)
        coeff = uc_t + ic_t + gc_t
        return acc + score * coeff

      acc = lax.fori_loop(0, _T, tag, jnp.zeros((_D,), jnp.float32))
      outv[pl.ds(c * _CH + g16, _D)] = acc

  fire(0)
  for c in range(_NCH):
    if c + 1 < _NCH:
      fire(c + 1)
    for cp in descs[c % 2]:
      cp.wait()
    compute(c)
  pltpu.sync_copy(outv, out_ref.at[pl.ds(base, _EPW)])


@jax.jit
def _run(user, item, uemb, iemb, upb, ipb, gpb, unb, inb, gnb, uc, ic, gc):
  mesh = plsc.VectorSubcoreMesh(core_axis_name="c", subcore_axis_name="s",
                                num_cores=_NC, num_subcores=_NS)
  bufset = [
      pltpu.VMEM((_CH,), jnp.int32),
      pltpu.VMEM((_CH,), jnp.int32),
      pltpu.VMEM((_CH,), jnp.int32),
      pltpu.VMEM((_CH,), jnp.int32),
      pltpu.VMEM((3 * _CH,), jnp.int32),
      pltpu.VMEM((3 * _CH,), jnp.int32),
      pltpu.VMEM((_CH, _ROW), jnp.float32),
      pltpu.VMEM((_CH, _ROW), jnp.float32),
      pltpu.VMEM((3 * _CH, _D), jnp.float32),
      pltpu.VMEM((3 * _CH, _D), jnp.float32),
      pltpu.VMEM((3 * _CH, _D), jnp.float32),
      pltpu.VMEM((3 * _CH, _D), jnp.float32),
      pltpu.VMEM((3 * _CH, _D), jnp.float32),
      pltpu.VMEM((3 * _CH, _D), jnp.float32),
  ]
  f = pl.kernel(
      _body,
      out_type=jax.ShapeDtypeStruct((_B,), jnp.float32),
      mesh=mesh,
      compiler_params=pltpu.CompilerParams(needs_layout_passes=False,
                                           use_tc_tiling_on_sc=False),
      scratch_types=bufset + bufset + [
          pltpu.VMEM((32,), jnp.float32),
          pltpu.VMEM((32,), jnp.float32),
          pltpu.VMEM((32,), jnp.float32),
          pltpu.VMEM((_CH * 32,), jnp.float32),
          pltpu.VMEM((_EPW,), jnp.float32),
          pltpu.SemaphoreType.DMA,
          pltpu.SemaphoreType.DMA,
      ],
  )
  return f(user, item, uemb, iemb, upb, ipb, gpb, unb, inb, gnb, uc, ic, gc)


def kernel(user, item, user_tag_embeddings, item_tag_embeddings,
           user_pos_bias, item_pos_bias, global_pos_bias,
           user_neg_bias, item_neg_bias, global_neg_bias,
           user_coeff, item_coeff, global_coeff):
  uemb = user_tag_embeddings.reshape(_UN, _ROW)
  iemb = item_tag_embeddings.reshape(_IN, _ROW)
  return _run(user.astype(jnp.int32), item.astype(jnp.int32), uemb, iemb,
              user_pos_bias.reshape(_FB, _D), item_pos_bias.reshape(_FB, _D),
              global_pos_bias.reshape(_T),
              user_neg_bias.reshape(_FB, _D), item_neg_bias.reshape(_FB, _D),
              global_neg_bias.reshape(_T),
              user_coeff.reshape(_FB, _D), item_coeff.reshape(_FB, _D),
              global_coeff.reshape(_T))


# speedup vs baseline: 1.8674x; 1.8674x over previous
import jax
import jax.numpy as jnp
from jax import lax
from jax.experimental import pallas as pl
from jax.experimental.pallas import tpu as pltpu
from jax.experimental.pallas import tpu_sc as plsc

_UN = 100000
_IN = 100000
_T = 26
_D = 16
_B = 4096
_ROW = _T * _D
_FB = _UN * _T // _D

_NC = 2
_NS = 16
_NW = _NC * _NS
_EPW = _B // _NW
_CH = 32
_NCH = _EPW // _CH
_NG = _CH // _D


def _body(user_ref, item_ref, uemb_ref, iemb_ref,
          upb_ref, ipb_ref, gpb_ref, unb_ref, inb_ref, gnb_ref,
          uc_ref, ic_ref, gc_ref, out_ref, *scr):
  nper = 14
  sets = [scr[:nper], scr[nper:2 * nper]]
  gpb_v, gnb_v, gc_v, dots, outv, sem0, sem1 = scr[2 * nper:]
  sems = [sem0, sem1]

  cid = lax.axis_index("c")
  sid = lax.axis_index("s")
  wid = sid * _NC + cid
  base = wid * _EPW

  pltpu.sync_copy(gpb_ref, gpb_v.at[pl.ds(0, _T)])
  pltpu.sync_copy(gnb_ref, gnb_v.at[pl.ds(0, _T)])
  pltpu.sync_copy(gc_ref, gc_v.at[pl.ds(0, _T)])

  lane = lax.iota(jnp.int32, _D)
  lane3 = lane * 3
  lane32 = lane * 32
  lane48 = lane * 48
  lane15 = lane == 15

  descs = [None, None]

  def fire(c):
    s = c % 2
    (uidx, iidx, offu, offi, b3u, b3i, urow, irow,
     upb, ipb, unb, inb, uc, ic) = sets[s]
    pltpu.sync_copy(user_ref.at[pl.ds(base + c * _CH, _CH)], uidx)
    pltpu.sync_copy(item_ref.at[pl.ds(base + c * _CH, _CH)], iidx)
    for k in range(_NG):
      pos = lane3 + (k * _D * 3)
      for vec, offbuf, b3buf in ((uidx, offu, b3u), (iidx, offi, b3i)):
        w = vec[pl.ds(k * _D, _D)] * _T
        r0 = lax.shift_right_logical(w, 4)
        offbuf[pl.ds(k * _D, _D)] = jnp.bitwise_and(w, 15)
        for j in range(3):
          rj = jnp.minimum(r0 + j, _FB - 1)
          plsc.store_scatter(b3buf, [pos + j], rj)
    descs[s] = [
        pltpu.async_copy(uemb_ref.at[uidx], urow, sems[s]),
        pltpu.async_copy(iemb_ref.at[iidx], irow, sems[s]),
        pltpu.async_copy(upb_ref.at[b3u], upb, sems[s]),
        pltpu.async_copy(unb_ref.at[b3u], unb, sems[s]),
        pltpu.async_copy(uc_ref.at[b3u], uc, sems[s]),
        pltpu.async_copy(ipb_ref.at[b3i], ipb, sems[s]),
        pltpu.async_copy(inb_ref.at[b3i], inb, sems[s]),
        pltpu.async_copy(ic_ref.at[b3i], ic, sems[s]),
    ]

  def compute(c):
    s = c % 2
    (uidx, iidx, offu, offi, b3u, b3i, urow, irow,
     upb, ipb, unb, inb, uc, ic) = sets[s]

    def elem(e, carry):
      for t in range(_T):
        u = urow[e, pl.ds(t * _D, _D)]
        w = irow[e, pl.ds(t * _D, _D)]
        prod = plsc.cumsum(u * w)
        slot = e * 32 + jnp.bitwise_and(e + t, 31)
        plsc.store_scatter(dots, [jnp.full((_D,), slot, jnp.int32)], prod,
                           mask=lane15)
      return carry

    lax.fori_loop(0, _CH, elem, 0)

    for g in range(_NG):
      g16 = g * _D
      dbase = lane32 + g16 * 32
      skew = lane + g16
      peru = lane48 + g16 * 48 + offu[pl.ds(g16, _D)]
      peri = lane48 + g16 * 48 + offi[pl.ds(g16, _D)]

      def tag(t, acc):
        tf = jnp.full((_D,), t, jnp.int32)
        d = plsc.load_gather(dots, [dbase + jnp.bitwise_and(skew + t, 31)])
        fu = peru + t
        fi = peri + t
        upb_t = plsc.load_gather(upb, [lax.shift_right_logical(fu, 4),
                                       jnp.bitwise_and(fu, 15)])
        unb_t = plsc.load_gather(unb, [lax.shift_right_logical(fu, 4),
                                       jnp.bitwise_and(fu, 15)])
        uc_t = plsc.load_gather(uc, [lax.shift_right_logical(fu, 4),
                                     jnp.bitwise_and(fu, 15)])
        ipb_t = plsc.load_gather(ipb, [lax.shift_right_logical(fi, 4),
                                       jnp.bitwise_and(fi, 15)])
        inb_t = plsc.load_gather(inb, [lax.shift_right_logical(fi, 4),
                                       jnp.bitwise_and(fi, 15)])
        ic_t = plsc.load_gather(ic, [lax.shift_right_logical(fi, 4),
                                     jnp.bitwise_and(fi, 15)])
        gp_t = plsc.load_gather(gpb_v, [tf])
        gn_t = plsc.load_gather(gnb_v, [tf])
        gc_t = plsc.load_gather(gc_v, [tf])

        one = jnp.float32(1.0)
        pos = one / (one + jnp.exp(-(d + upb_t + ipb_t + gp_t)))
        neg = one / (one + jnp.exp(-(d + unb_t + inb_t + gn_t)))
        score = (pos + neg) * jnp.float32(0.5)
        coeff = uc_t + ic_t + gc_t
        return acc + score * coeff

      acc = lax.fori_loop(0, _T, tag, jnp.zeros((_D,), jnp.float32))
      outv[pl.ds(c * _CH + g16, _D)] = acc

  fire(0)
  for c in range(_NCH):
    if c + 1 < _NCH:
      fire(c + 1)
    for cp in descs[c % 2]:
      cp.wait()
    compute(c)
  pltpu.sync_copy(outv, out_ref.at[pl.ds(base, _EPW)])


@jax.jit
def _run(user, item, uemb, iemb, upb, ipb, gpb, unb, inb, gnb, uc, ic, gc):
  mesh = plsc.VectorSubcoreMesh(core_axis_name="c", subcore_axis_name="s",
                                num_cores=_NC, num_subcores=_NS)
  bufset = [
      pltpu.VMEM((_CH,), jnp.int32),
      pltpu.VMEM((_CH,), jnp.int32),
      pltpu.VMEM((_CH,), jnp.int32),
      pltpu.VMEM((_CH,), jnp.int32),
      pltpu.VMEM((3 * _CH,), jnp.int32),
      pltpu.VMEM((3 * _CH,), jnp.int32),
      pltpu.VMEM((_CH, _ROW), jnp.float32),
      pltpu.VMEM((_CH, _ROW), jnp.float32),
      pltpu.VMEM((3 * _CH, _D), jnp.float32),
      pltpu.VMEM((3 * _CH, _D), jnp.float32),
      pltpu.VMEM((3 * _CH, _D), jnp.float32),
      pltpu.VMEM((3 * _CH, _D), jnp.float32),
      pltpu.VMEM((3 * _CH, _D), jnp.float32),
      pltpu.VMEM((3 * _CH, _D), jnp.float32),
  ]
  f = pl.kernel(
      _body,
      out_type=jax.ShapeDtypeStruct((_B,), jnp.float32),
      mesh=mesh,
      compiler_params=pltpu.CompilerParams(needs_layout_passes=False,
                                           use_tc_tiling_on_sc=False),
      scratch_types=bufset + bufset + [
          pltpu.VMEM((32,), jnp.float32),
          pltpu.VMEM((32,), jnp.float32),
          pltpu.VMEM((32,), jnp.float32),
          pltpu.VMEM((_CH * 32,), jnp.float32),
          pltpu.VMEM((_EPW,), jnp.float32),
          pltpu.SemaphoreType.DMA,
          pltpu.SemaphoreType.DMA,
      ],
  )
  return f(user, item, uemb, iemb, upb, ipb, gpb, unb, inb, gnb, uc, ic, gc)


def kernel(user, item, user_tag_embeddings, item_tag_embeddings,
           user_pos_bias, item_pos_bias, global_pos_bias,
           user_neg_bias, item_neg_bias, global_neg_bias,
           user_coeff, item_coeff, global_coeff):
  eye = jnp.eye(_ROW, dtype=jnp.float32)
  uemb = user_tag_embeddings.reshape(_UN, _ROW) @ eye
  iemb = item_tag_embeddings.reshape(_IN, _ROW) @ eye
  return _run(user.astype(jnp.int32), item.astype(jnp.int32), uemb, iemb,
              user_pos_bias.reshape(_FB, _D), item_pos_bias.reshape(_FB, _D),
              global_pos_bias.reshape(_T),
              user_neg_bias.reshape(_FB, _D), item_neg_bias.reshape(_FB, _D),
              global_neg_bias.reshape(_T),
              user_coeff.reshape(_FB, _D), item_coeff.reshape(_FB, _D),
              global_coeff.reshape(_T))
